# baseline (speedup 1.0000x reference)
import functools

import jax
import jax.numpy as jnp
from jax import lax
from jax.experimental import pallas as pl
from jax.experimental.pallas import tpu as pltpu

NUM_SEGS = 64
ROWS = 100000
BLK1 = 20000
NB1 = ROWS // BLK1
BLK2 = 10000
NB2 = ROWS // BLK2
EPS = 1e-8


def _fused_body(batch1_ref, batch2_ref, x_ref, w_ref, bia_ref, ms_ref,
                out_ref, xs_ref, sums_ref, sqs_ref, cnts_ref):
    i = pl.program_id(0)

    @pl.when(i == 0)
    def _init():
        sums_ref[...] = jnp.zeros_like(sums_ref)
        sqs_ref[...] = jnp.zeros_like(sqs_ref)
        cnts_ref[...] = jnp.zeros_like(cnts_ref)

    @pl.when(i < NB1)
    def _phase_stats():
        b = batch1_ref[0]
        seg_ids = lax.broadcasted_iota(jnp.int32, (NUM_SEGS, BLK1), 0)
        oht = (jnp.broadcast_to(b, (NUM_SEGS, BLK1)) == seg_ids
               ).astype(jnp.bfloat16)
        xb = x_ref[...]
        xb16 = xb.astype(jnp.bfloat16)
        xs_ref[pl.ds(i * BLK1, BLK1), :] = xb16
        sq16 = (xb * xb).astype(jnp.bfloat16)
        dn = (((1,), (0,)), ((), ()))
        sums_ref[...] += lax.dot_general(oht, xb16, dn,
                                         preferred_element_type=jnp.float32)
        sqs_ref[...] += lax.dot_general(oht, sq16, dn,
                                        preferred_element_type=jnp.float32)
        cnts_ref[...] += jnp.broadcast_to(
            jnp.sum(oht.astype(jnp.float32), axis=1).reshape(NUM_SEGS, 1),
            (NUM_SEGS, 128))

    @pl.when(i >= NB1)
    def _phase_norm():
        j = i - NB1
        cnt = jnp.maximum(cnts_ref[...], 1.0)
        mean = sums_ref[...] / cnt
        var = (sqs_ref[...] - cnt * mean * mean) / jnp.maximum(cnt - 1.0, 1.0)
        std = jnp.sqrt(jnp.maximum(var, 0.0)) + EPS
        a = w_ref[...] / std
        bcoef = bia_ref[...] - a * mean * ms_ref[...]

        b = batch2_ref[0]
        seg_ids = lax.broadcasted_iota(jnp.int32, (BLK2, NUM_SEGS), 1)
        oh = (jnp.broadcast_to(b.reshape(BLK2, 1), (BLK2, NUM_SEGS)) == seg_ids
              ).astype(jnp.bfloat16)
        ab = jnp.concatenate([a, bcoef], axis=1).astype(jnp.bfloat16)
        dn = (((1,), (0,)), ((), ()))
        ab_rows = lax.dot_general(oh, ab, dn,
                                  preferred_element_type=jnp.float32)
        xb = xs_ref[pl.ds(j * BLK2, BLK2), :].astype(jnp.float32)
        out_ref[...] = xb * ab_rows[:, :128] + ab_rows[:, 128:]


@functools.partial(jax.jit, static_argnames=("interpret",))
def kernel(x, batch, weight, bias, mean_scale, interpret=False):
    batch_i32 = batch.astype(jnp.int32)
    batch1 = batch_i32.reshape(NB1, 1, BLK1)
    batch2 = batch_i32.reshape(NB2, 1, BLK2)
    out = pl.pallas_call(
        _fused_body,
        grid=(NB1 + NB2,),
        in_specs=[
            pl.BlockSpec((1, 1, BLK1),
                         lambda i: (jnp.minimum(i, NB1 - 1), 0, 0)),
            pl.BlockSpec((1, 1, BLK2),
                         lambda i: (jnp.maximum(i - NB1, 0), 0, 0)),
            pl.BlockSpec((BLK1, 128), lambda i: (jnp.minimum(i, NB1 - 1), 0)),
            pl.BlockSpec((1, 128), lambda i: (0, 0)),
            pl.BlockSpec((1, 128), lambda i: (0, 0)),
            pl.BlockSpec((1, 128), lambda i: (0, 0)),
        ],
        out_specs=pl.BlockSpec((BLK2, 128),
                               lambda i: (jnp.maximum(i - NB1, 0), 0)),
        out_shape=jax.ShapeDtypeStruct((ROWS, 128), jnp.float32),
        scratch_shapes=[
            pltpu.VMEM((ROWS, 128), jnp.bfloat16),
            pltpu.VMEM((NUM_SEGS, 128), jnp.float32),
            pltpu.VMEM((NUM_SEGS, 128), jnp.float32),
            pltpu.VMEM((NUM_SEGS, 128), jnp.float32),
        ],
        interpret=interpret,
    )(batch1, batch2, x, weight.reshape(1, 128), bias.reshape(1, 128),
      mean_scale.reshape(1, 128))
    return out


# speedup vs baseline: 1.0291x; 1.0291x over previous
import functools

import jax
import jax.numpy as jnp
from jax import lax
from jax.experimental import pallas as pl
from jax.experimental.pallas import tpu as pltpu

NUM_SEGS = 64
ROWS = 100000
BLK = 10000
NB = ROWS // BLK
EPS = 1e-8


def _fused_body(batch_ref, x_ref, w_ref, bia_ref, ms_ref, out_ref,
                xs_ref, sums_ref, sqs_ref, cnts_ref):
    i = pl.program_id(0)

    @pl.when(i == 0)
    def _init():
        sums_ref[...] = jnp.zeros_like(sums_ref)
        sqs_ref[...] = jnp.zeros_like(sqs_ref)
        cnts_ref[...] = jnp.zeros_like(cnts_ref)

    @pl.when(i < NB)
    def _phase_stats():
        b = batch_ref[0]
        seg_ids = lax.broadcasted_iota(jnp.int32, (NUM_SEGS, BLK), 0)
        oht = (jnp.broadcast_to(b, (NUM_SEGS, BLK)) == seg_ids
               ).astype(jnp.bfloat16)
        xb = x_ref[...]
        xb16 = xb.astype(jnp.bfloat16)
        xs_ref[pl.ds(i * BLK, BLK), :] = xb16
        sq16 = (xb * xb).astype(jnp.bfloat16)
        dn = (((1,), (0,)), ((), ()))
        sums_ref[...] += lax.dot_general(oht, xb16, dn,
                                         preferred_element_type=jnp.float32)
        sqs_ref[...] += lax.dot_general(oht, sq16, dn,
                                        preferred_element_type=jnp.float32)
        cnts_ref[...] += jnp.broadcast_to(
            jnp.sum(oht.astype(jnp.float32), axis=1).reshape(NUM_SEGS, 1),
            (NUM_SEGS, 128))

    @pl.when(i >= NB)
    def _phase_norm():
        j = i - NB
        cnt = jnp.maximum(cnts_ref[...], 1.0)
        mean = sums_ref[...] / cnt
        var = (sqs_ref[...] - cnt * mean * mean) / jnp.maximum(cnt - 1.0, 1.0)
        std = jnp.sqrt(jnp.maximum(var, 0.0)) + EPS
        a = w_ref[...] / std
        bcoef = bia_ref[...] - a * mean * ms_ref[...]

        b = batch_ref[0]
        seg_ids = lax.broadcasted_iota(jnp.int32, (BLK, NUM_SEGS), 1)
        oh = (jnp.broadcast_to(b.reshape(BLK, 1), (BLK, NUM_SEGS)) == seg_ids
              ).astype(jnp.bfloat16)
        ab = jnp.concatenate([a, bcoef], axis=1).astype(jnp.bfloat16)
        dn = (((1,), (0,)), ((), ()))
        ab_rows = lax.dot_general(oh, ab, dn,
                                  preferred_element_type=jnp.float32)
        xb = xs_ref[pl.ds(j * BLK, BLK), :].astype(jnp.float32)
        out_ref[...] = xb * ab_rows[:, :128] + ab_rows[:, 128:]


@functools.partial(jax.jit, static_argnames=("interpret",))
def kernel(x, batch, weight, bias, mean_scale, interpret=False):
    batch3 = batch.astype(jnp.int32).reshape(NB, 1, BLK)
    out = pl.pallas_call(
        _fused_body,
        grid=(2 * NB,),
        in_specs=[
            pl.BlockSpec((1, 1, BLK),
                         lambda i: (jnp.where(i < NB, i, i - NB), 0, 0)),
            pl.BlockSpec((BLK, 128), lambda i: (jnp.minimum(i, NB - 1), 0)),
            pl.BlockSpec((1, 128), lambda i: (0, 0)),
            pl.BlockSpec((1, 128), lambda i: (0, 0)),
            pl.BlockSpec((1, 128), lambda i: (0, 0)),
        ],
        out_specs=pl.BlockSpec((BLK, 128),
                               lambda i: (jnp.where(i < NB, 0, i - NB), 0)),
        out_shape=jax.ShapeDtypeStruct((ROWS, 128), jnp.float32),
        scratch_shapes=[
            pltpu.VMEM((ROWS, 128), jnp.bfloat16),
            pltpu.VMEM((NUM_SEGS, 128), jnp.float32),
            pltpu.VMEM((NUM_SEGS, 128), jnp.float32),
            pltpu.VMEM((NUM_SEGS, 128), jnp.float32),
        ],
        interpret=interpret,
    )(batch3, x, weight.reshape(1, 128), bias.reshape(1, 128),
      mean_scale.reshape(1, 128))
    return out
